# baseline (speedup 1.0000x reference)
import functools

import jax
import jax.numpy as jnp
from jax import lax
from jax.experimental import pallas as pl
from jax.experimental.pallas import tpu as pltpu
from jax.experimental.pallas import tpu_sc as plsc

_NUM_REG = 4
_NUM_CAT = 26
_VOCAB = 100000
_DIM = 32
_BATCH = 16384
_NF = _NUM_REG + _NUM_CAT

_NC = 2
_NS = 16
_NW = _NC * _NS
_L = 16

_B_PER_W = _BATCH // _NW
_NB = 64
_NCHUNK = _B_PER_W // _NB
_ROWS = _NB * _NF


def _body(inp_hbm, tables_hbm, wreg_hbm, breg_hbm, out_hbm,
          inp_v, idx_v, rows_v, wreg_v, breg_v, sem):
    wid = lax.axis_index("s") * _NC + lax.axis_index("c")

    pltpu.sync_copy(wreg_hbm, wreg_v)
    pltpu.sync_copy(breg_hbm, breg_v)

    lane = lax.broadcasted_iota(jnp.int32, (_L,), 0)

    def chunk(ck, carry):
        p0 = (wid * _B_PER_W + ck * _NB) * _NF
        pltpu.sync_copy(inp_hbm.at[pl.ds(p0, _ROWS)], inp_v)

        def ivec(k, c):
            base = k * _L
            p = base + lane
            i = lax.rem(p, _NF)
            vals = inp_v[pl.ds(base + _NUM_REG, _L)]
            idx = jnp.where(i < _NUM_CAT, i * _VOCAB + vals.astype(jnp.int32), 0)
            idx_v[pl.ds(base, _L)] = idx
            return c

        lax.fori_loop(0, _ROWS // _L, ivec, 0)

        pltpu.async_copy(tables_hbm.at[idx_v], rows_v, sem).wait()

        def regrow(b, c):
            xs = inp_v[pl.ds(b * _NF, _L)]
            for j in range(_NUM_REG):
                x = xs[j]
                r = b * _NF + _NUM_CAT + j
                rows_v[r, pl.ds(0, _L)] = x * wreg_v[j, pl.ds(0, _L)] + breg_v[j, pl.ds(0, _L)]
                rows_v[r, pl.ds(_L, _L)] = x * wreg_v[j, pl.ds(_L, _L)] + breg_v[j, pl.ds(_L, _L)]
            return c

        lax.fori_loop(0, _NB, regrow, 0)

        pltpu.sync_copy(rows_v, out_hbm.at[pl.ds(p0, _ROWS)])
        return carry

    lax.fori_loop(0, _NCHUNK, chunk, 0)


@jax.jit
def kernel(all_inputs, tables, Wreg, breg):
    inp_flat = all_inputs.reshape(_BATCH * _NF)
    tables_flat = tables.reshape(_NUM_CAT * _VOCAB, _DIM)

    mesh = plsc.VectorSubcoreMesh(core_axis_name="c", subcore_axis_name="s")
    out = pl.kernel(
        _body,
        out_type=jax.ShapeDtypeStruct((_BATCH * _NF, _DIM), jnp.float32),
        mesh=mesh,
        scratch_types=[
            pltpu.VMEM((_ROWS,), jnp.float32),
            pltpu.VMEM((_ROWS,), jnp.int32),
            pltpu.VMEM((_ROWS, _DIM), jnp.float32),
            pltpu.VMEM((_NUM_REG, _DIM), jnp.float32),
            pltpu.VMEM((_NUM_REG, _DIM), jnp.float32),
            pltpu.SemaphoreType.DMA,
        ],
        compiler_params=pltpu.CompilerParams(use_tc_tiling_on_sc=False),
    )(inp_flat, tables_flat, Wreg, breg)
    return out.reshape(_BATCH, _NF, _DIM)
